# baseline (speedup 1.0000x reference)
import functools

import jax
import jax.numpy as jnp
from jax import lax
from jax.experimental import pallas as pl
from jax.experimental.pallas import tpu as pltpu
from jax.experimental.pallas import tpu_sc as plsc

_INFO = plsc.get_sparse_core_info()
_NC = _INFO.num_cores
_NS = _INFO.num_subcores
_NW = _NC * _NS


def _project_table(emb, W, b2d):
    V, D = emb.shape[0], W.shape[1]

    def body(emb_ref, w_ref, b_ref, out_ref):
        out_ref[...] = (
            jnp.dot(emb_ref[...], w_ref[...], preferred_element_type=jnp.float32)
            + b_ref[...]
        )

    return pl.pallas_call(
        body,
        out_shape=jax.ShapeDtypeStruct((V, D), jnp.float32),
    )(emb, W, b2d)


def _make_gather(B, D, V, chunk, nbuf):
    b_per_w = B // _NW
    n_ch = b_per_w // chunk
    n_grp = n_ch // nbuf
    mesh = plsc.VectorSubcoreMesh(core_axis_name="c", subcore_axis_name="s")

    @functools.partial(
        pl.kernel,
        mesh=mesh,
        out_type=jax.ShapeDtypeStruct((B, D), jnp.float32),
        scratch_types=(
            [pltpu.VMEM_SHARED((V, D), jnp.float32),
             pltpu.VMEM((b_per_w,), jnp.int32)]
            + [pltpu.VMEM((chunk, D), jnp.float32) for _ in range(nbuf)]
            + [pltpu.SemaphoreType.DMA for _ in range(2 * nbuf)]
        ),
    )
    def gather_kernel(table_hbm, idx_hbm, out_hbm, table_sp, idx_all, *bufs):
        rows = bufs[:nbuf]
        gsem = bufs[nbuf:2 * nbuf]
        wsem = bufs[2 * nbuf:]
        wid = lax.axis_index("s") * _NC + lax.axis_index("c")
        base = wid * b_per_w

        @pl.when(lax.axis_index("s") == 0)
        def _stage_table():
            pltpu.sync_copy(table_hbm, table_sp)

        pltpu.sync_copy(idx_hbm.at[pl.ds(base, b_per_w)], idx_all)
        plsc.subcore_barrier()

        def gather_issue(c, bf):
            for g in range(chunk // 16):
                vec = idx_all[pl.ds(c * chunk + g * 16, 16)]
                for j in range(16):
                    pltpu.async_copy(
                        table_sp.at[vec[j]], rows[bf].at[g * 16 + j],
                        gsem[bf])

        def gather_wait(bf):
            pltpu.make_async_copy(
                out_hbm.at[pl.ds(0, chunk)], rows[bf], gsem[bf]).wait()

        def write_issue(c, bf):
            pltpu.async_copy(
                rows[bf], out_hbm.at[pl.ds(base + c * chunk, chunk)], wsem[bf])

        def write_wait(bf):
            pltpu.make_async_copy(
                rows[bf], out_hbm.at[pl.ds(0, chunk)], wsem[bf]).wait()

        for bf in range(nbuf):
            gather_issue(bf, bf)

        def body(p, carry):
            c0 = p * nbuf
            for bf in range(nbuf):
                gather_wait(bf)
                write_issue(c0 + bf, bf)

            @pl.when(p < n_grp - 1)
            def _prefetch():
                for bf in range(nbuf):
                    write_wait(bf)
                    gather_issue(c0 + nbuf + bf, bf)

            return carry

        lax.fori_loop(0, n_grp, body, 0)
        for bf in range(nbuf):
            write_wait(bf)

    return gather_kernel


def kernel(actions, embedding_table, W, b):
    Bb, L = actions.shape
    D = W.shape[1]
    B = Bb * L
    V = embedding_table.shape[0]

    table = _project_table(embedding_table, W, b.reshape(1, D))
    idx = actions.T.reshape(B)

    chunk, nbuf = 16, 4
    out = _make_gather(B, D, V, chunk, nbuf)(table, idx)
    return out.reshape(L, Bb, D).transpose(1, 0, 2)


# speedup vs baseline: 1.1447x; 1.1447x over previous
import functools

import jax
import jax.numpy as jnp
from jax import lax
from jax.experimental import pallas as pl
from jax.experimental.pallas import tpu as pltpu
from jax.experimental.pallas import tpu_sc as plsc

_INFO = plsc.get_sparse_core_info()
_NC = _INFO.num_cores
_NS = _INFO.num_subcores
_NW = _NC * _NS


def _project_table(emb, W, b2d):
    V, D = emb.shape[0], W.shape[1]

    def body(emb_ref, w_ref, b_ref, out_ref):
        out_ref[...] = (
            jnp.dot(emb_ref[...], w_ref[...], preferred_element_type=jnp.float32)
            + b_ref[...]
        )

    return pl.pallas_call(
        body,
        out_shape=jax.ShapeDtypeStruct((V, D), jnp.float32),
    )(emb, W, b2d)


def _make_gather(B, D, V, chunk):
    b_per_w = B // _NW
    n_ch = b_per_w // chunk
    n_pairs = n_ch // 2
    mesh = plsc.VectorSubcoreMesh(core_axis_name="c", subcore_axis_name="s")

    @functools.partial(
        pl.kernel,
        mesh=mesh,
        out_type=jax.ShapeDtypeStruct((B, D), jnp.float32),
        scratch_types=[
            pltpu.VMEM_SHARED((V, D), jnp.float32),
            pltpu.VMEM((b_per_w,), jnp.int32),
            pltpu.VMEM((chunk, D), jnp.float32),
            pltpu.VMEM((chunk, D), jnp.float32),
            pltpu.SemaphoreType.DMA,
            pltpu.SemaphoreType.DMA,
            pltpu.SemaphoreType.DMA,
            pltpu.SemaphoreType.DMA,
        ],
    )
    def gather_kernel(table_hbm, idx_hbm, out_hbm, table_sp, idx_all,
                      rows0, rows1, gsem0, gsem1, wsem0, wsem1):
        wid = lax.axis_index("s") * _NC + lax.axis_index("c")
        base = wid * b_per_w

        rows = (rows0, rows1)
        gsem = (gsem0, gsem1)
        wsem = (wsem0, wsem1)

        @pl.when(lax.axis_index("s") == 0)
        def _stage_table():
            pltpu.sync_copy(table_hbm, table_sp)

        pltpu.sync_copy(idx_hbm.at[pl.ds(base, b_per_w)], idx_all)
        plsc.subcore_barrier()

        def gather_issue(c, bf):
            for g in range(chunk // 16):
                vec = idx_all[pl.ds(c * chunk + g * 16, 16)]
                for j in range(16):
                    pltpu.async_copy(
                        table_sp.at[vec[j]], rows[bf].at[g * 16 + j],
                        gsem[bf])

        def gather_wait(bf):
            pltpu.make_async_copy(
                out_hbm.at[pl.ds(0, chunk)], rows[bf], gsem[bf]).wait()

        def write_issue(c, bf):
            pltpu.async_copy(
                rows[bf], out_hbm.at[pl.ds(base + c * chunk, chunk)], wsem[bf])

        def write_wait(bf):
            pltpu.make_async_copy(
                rows[bf], out_hbm.at[pl.ds(0, chunk)], wsem[bf]).wait()

        gather_issue(0, 0)
        gather_issue(1, 1)

        def body(q, carry):
            for r in range(2):
                p = 2 * q + r
                c0 = 2 * p
                gather_wait(0)
                write_issue(c0, 0)
                gather_wait(1)
                write_issue(c0 + 1, 1)

                @pl.when(p < n_pairs - 1)
                def _prefetch():
                    write_wait(0)
                    gather_issue(c0 + 2, 0)
                    write_wait(1)
                    gather_issue(c0 + 3, 1)

            return carry

        lax.fori_loop(0, n_pairs // 2, body, 0)
        write_wait(0)
        write_wait(1)

    return gather_kernel


def kernel(actions, embedding_table, W, b):
    Bb, L = actions.shape
    D = W.shape[1]
    B = Bb * L
    V = embedding_table.shape[0]

    table = _project_table(embedding_table, W, b.reshape(1, D))
    idx = actions.T.reshape(B)

    chunk = 16
    out = _make_gather(B, D, V, chunk)(table, idx)
    return out.reshape(L, Bb, D).transpose(1, 0, 2)


# speedup vs baseline: 1.1460x; 1.0012x over previous
import functools

import jax
import jax.numpy as jnp
from jax import lax
from jax.experimental import pallas as pl
from jax.experimental.pallas import tpu as pltpu
from jax.experimental.pallas import tpu_sc as plsc

_INFO = plsc.get_sparse_core_info()
_NC = _INFO.num_cores
_NS = _INFO.num_subcores
_NW = _NC * _NS


def _project_table(emb, W, b2d):
    V, D = emb.shape[0], W.shape[1]

    def body(emb_ref, w_ref, b_ref, out_ref):
        out_ref[...] = (
            jnp.dot(emb_ref[...], w_ref[...], preferred_element_type=jnp.float32)
            + b_ref[...]
        )

    return pl.pallas_call(
        body,
        out_shape=jax.ShapeDtypeStruct((V, D), jnp.float32),
    )(emb, W, b2d)


def _make_gather(B, D, V, chunk):
    b_per_w = B // _NW
    n_ch = b_per_w // chunk
    n_pairs = n_ch // 2
    mesh = plsc.VectorSubcoreMesh(core_axis_name="c", subcore_axis_name="s")

    @functools.partial(
        pl.kernel,
        mesh=mesh,
        out_type=jax.ShapeDtypeStruct((B, D), jnp.float32),
        scratch_types=[
            pltpu.VMEM_SHARED((V, D), jnp.float32),
            pltpu.VMEM((b_per_w,), jnp.int32),
            pltpu.VMEM((chunk, D), jnp.float32),
            pltpu.VMEM((chunk, D), jnp.float32),
            pltpu.SemaphoreType.DMA,
            pltpu.SemaphoreType.DMA,
            pltpu.SemaphoreType.DMA,
            pltpu.SemaphoreType.DMA,
        ],
    )
    def gather_kernel(table_hbm, idx_hbm, out_hbm, table_sp, idx_all,
                      rows0, rows1, gsem0, gsem1, wsem0, wsem1):
        wid = lax.axis_index("s") * _NC + lax.axis_index("c")
        base = wid * b_per_w

        rows = (rows0, rows1)
        gsem = (gsem0, gsem1)
        wsem = (wsem0, wsem1)

        @pl.when(lax.axis_index("s") == 0)
        def _stage_table():
            pltpu.sync_copy(table_hbm, table_sp)

        pltpu.sync_copy(idx_hbm.at[pl.ds(base, b_per_w)], idx_all)
        plsc.subcore_barrier()

        def gather_issue(c, bf):
            for g in range(chunk // 16):
                vec = idx_all[pl.ds(c * chunk + g * 16, 16)]
                for j in range(16):
                    pltpu.async_copy(
                        table_sp.at[vec[j]], rows[bf].at[g * 16 + j],
                        gsem[bf])

        def gather_wait(bf):
            pltpu.make_async_copy(
                out_hbm.at[pl.ds(0, chunk)], rows[bf], gsem[bf]).wait()

        def write_issue(c, bf):
            pltpu.async_copy(
                rows[bf], out_hbm.at[pl.ds(base + c * chunk, chunk)], wsem[bf])

        def write_wait(bf):
            pltpu.make_async_copy(
                rows[bf], out_hbm.at[pl.ds(0, chunk)], wsem[bf]).wait()

        gather_issue(0, 0)
        gather_issue(1, 1)

        def body(p, carry):
            c0 = 2 * p
            gather_wait(0)
            write_issue(c0, 0)
            gather_wait(1)
            write_issue(c0 + 1, 1)

            @pl.when(p < n_pairs - 1)
            def _prefetch():
                write_wait(0)
                gather_issue(c0 + 2, 0)
                write_wait(1)
                gather_issue(c0 + 3, 1)

            return carry

        lax.fori_loop(0, n_pairs, body, 0)
        write_wait(0)
        write_wait(1)

    return gather_kernel


def kernel(actions, embedding_table, W, b):
    Bb, L = actions.shape
    D = W.shape[1]
    B = Bb * L
    V = embedding_table.shape[0]

    table = _project_table(embedding_table, W, b.reshape(1, D))
    idx = actions.T.reshape(B)

    chunk = 16
    out = _make_gather(B, D, V, chunk)(table, idx)
    return out.reshape(L, Bb, D).transpose(1, 0, 2)
